# baseline (speedup 1.0000x reference)
import functools

import jax
import jax.numpy as jnp
from jax import lax
from jax.experimental import pallas as pl
from jax.experimental.pallas import tpu as pltpu
from jax.experimental.pallas import tpu_sc as plsc

N = 10000
NPAD = 10240
E = 320000
D = 128
H = 128
C = 64
G = 128

NC = 2
NS = 16
NW = NC * NS
EG = 128
E_ALL = E + N
G0 = 120
G1 = 56
NGRP = NS * (G0 + G1)
NGRP_PAD = NGRP + (G0 - G1)
E_PAD = NGRP * EG
ROWS_PER_SUB = NPAD // NS


def _zero_vmem_rows(buf, nrows, width):
    z = jnp.zeros((16,), jnp.float32)

    def body(i, _):
        for j in range(width // 16):
            buf[i, pl.ds(j * 16, 16)] = z
        return 0

    lax.fori_loop(0, nrows, body, 0)


def _fill_vmem_rows(buf, nrows, width, value):
    v = jnp.full((16,), value, jnp.float32)

    def body(i, _):
        for j in range(width // 16):
            buf[i, pl.ds(j * 16, 16)] = v
        return 0

    lax.fori_loop(0, nrows, body, 0)


def _copy_rows_to_shared(buf, acc_sh, base):
    full, rem = divmod(ROWS_PER_SUB, EG)
    for g in range(full):
        pltpu.sync_copy(buf, acc_sh.at[pl.ds(base + g * EG, EG)])
    if rem:
        pltpu.sync_copy(buf.at[pl.ds(0, rem)],
                        acc_sh.at[pl.ds(base + full * EG, rem)])


def _unpack_group(pk_v, j, sbuf, dbuf):
    for k in range(EG // 16):
        v = pk_v[j, pl.ds(k * 16, 16)]
        if sbuf is not None:
            sbuf[pl.ds(k * 16, 16)] = v & 0xFFFF
        dbuf[pl.ds(k * 16, 16)] = v >> 16


DGPT = NGRP // NW


def _deg_body(pk_hbm, out_hbm, pk_v, ones_v, dbuf, acc_sh):
    c = lax.axis_index("c")
    s = lax.axis_index("s")
    wid = s * NC + c

    _zero_vmem_rows(ones_v, EG, H)
    _copy_rows_to_shared(ones_v, acc_sh, s * ROWS_PER_SUB)
    _fill_vmem_rows(ones_v, EG, H, 1.0)
    plsc.subcore_barrier()

    pltpu.sync_copy(pk_hbm.at[pl.ds(wid * DGPT, DGPT)], pk_v)

    def body(j, _):
        _unpack_group(pk_v, j, None, dbuf)
        pltpu.sync_copy(ones_v, acc_sh.at[dbuf], add=True)
        return 0

    lax.fori_loop(0, DGPT, body, 0)
    plsc.subcore_barrier()

    pltpu.sync_copy(
        acc_sh.at[pl.ds(s * ROWS_PER_SUB, ROWS_PER_SUB)],
        out_hbm.at[c, pl.ds(s * ROWS_PER_SUB, ROWS_PER_SUB)],
    )


def _pipelined_groups(table_hbm, pk_v, sa, da, sb, db, rows0, rows1,
                      acc_sh, semA, semB, ngroups):
    _unpack_group(pk_v, 0, sa, da)
    pltpu.async_copy(table_hbm.at[sa], rows0, semA)

    def body(jj, _):
        j0 = 2 * jj
        _unpack_group(pk_v, j0 + 1, sb, db)
        pltpu.async_copy(table_hbm.at[sb], rows1, semB)
        pltpu.make_async_copy(table_hbm.at[pl.ds(0, EG)], rows0, semA).wait()
        pltpu.sync_copy(rows0, acc_sh.at[da], add=True)

        _unpack_group(pk_v, jnp.minimum(j0 + 2, ngroups - 1), sa, da)
        pltpu.async_copy(table_hbm.at[sa], rows0, semA)

        pltpu.make_async_copy(table_hbm.at[pl.ds(0, EG)], rows1, semB).wait()
        pltpu.sync_copy(rows1, acc_sh.at[db], add=True)
        return 0

    lax.fori_loop(0, ngroups // 2, body, 0)
    pltpu.make_async_copy(table_hbm.at[pl.ds(0, EG)], rows0, semA).wait()


def _serial_groups(table_hbm, pk_v, sa, da, rows0, acc_sh, semA, ngroups):
    def body(j, _):
        _unpack_group(pk_v, j, sa, da)
        pltpu.async_copy(table_hbm.at[sa], rows0, semA).wait()
        pltpu.sync_copy(rows0, acc_sh.at[da], add=True)
        return 0

    lax.fori_loop(0, ngroups, body, 0)


def _prop_body(table_hbm, pk_hbm, out_hbm, pk_v, sa, da, sb, db, rows0,
               rows1, acc_sh, semA, semB):
    c = lax.axis_index("c")
    s = lax.axis_index("s")

    _zero_vmem_rows(rows0, EG, H)
    _copy_rows_to_shared(rows0, acc_sh, s * ROWS_PER_SUB)
    plsc.subcore_barrier()

    base = jnp.where(c == 0, s * G0, NS * G0 + s * G1)
    pltpu.sync_copy(pk_hbm.at[pl.ds(base, G0)], pk_v)

    @pl.when(c == 0)
    def _():
        _pipelined_groups(table_hbm, pk_v, sa, da, sb, db, rows0, rows1,
                          acc_sh, semA, semB, G0)

    @pl.when(c == 1)
    def _():
        _serial_groups(table_hbm, pk_v, sa, da, rows0, acc_sh, semA, G1)

    plsc.subcore_barrier()

    pltpu.sync_copy(
        acc_sh.at[pl.ds(s * ROWS_PER_SUB, ROWS_PER_SUB)],
        out_hbm.at[c, pl.ds(s * ROWS_PER_SUB, ROWS_PER_SUB)],
    )


@functools.cache
def _sc_kernels():
    mesh = plsc.VectorSubcoreMesh(core_axis_name="c", subcore_axis_name="s")
    deg = pl.kernel(
        _deg_body,
        out_type=jax.ShapeDtypeStruct((NC, NPAD, H), jnp.float32),
        mesh=mesh,
        scratch_types=[
            pltpu.VMEM((DGPT, EG), jnp.int32),
            pltpu.VMEM((EG, H), jnp.float32),
            pltpu.VMEM((EG,), jnp.int32),
            pltpu.VMEM_SHARED((NPAD, H), jnp.float32),
        ],
    )
    prop = pl.kernel(
        _prop_body,
        out_type=jax.ShapeDtypeStruct((NC, NPAD, H), jnp.float32),
        mesh=mesh,
        scratch_types=[
            pltpu.VMEM((G0, EG), jnp.int32),
            pltpu.VMEM((EG,), jnp.int32),
            pltpu.VMEM((EG,), jnp.int32),
            pltpu.VMEM((EG,), jnp.int32),
            pltpu.VMEM((EG,), jnp.int32),
            pltpu.VMEM((EG, H), jnp.float32),
            pltpu.VMEM((EG, H), jnp.float32),
            pltpu.VMEM_SHARED((NPAD, H), jnp.float32),
            pltpu.SemaphoreType.DMA,
            pltpu.SemaphoreType.DMA,
        ],
    )
    return deg, prop


_BM = 1024
_GRID = NPAD // _BM


def _dinv_block(degb):
    deg = degb[0, :, 0:1] + degb[1, :, 0:1]
    return lax.rsqrt(jnp.maximum(deg, 1.0))


def _tc_first_body(xb, wb, degb, tableb):
    t = jnp.dot(xb[...], wb[...], preferred_element_type=jnp.float32)
    tableb[...] = t * _dinv_block(degb[...])


def _tc_first(x_pad, w, degp):
    return pl.pallas_call(
        _tc_first_body,
        grid=(_GRID,),
        in_specs=[
            pl.BlockSpec((_BM, D), lambda i: (i, 0)),
            pl.BlockSpec((D, H), lambda i: (0, 0)),
            pl.BlockSpec((NC, _BM, H), lambda i: (0, i, 0)),
        ],
        out_specs=pl.BlockSpec((_BM, H), lambda i: (i, 0)),
        out_shape=jax.ShapeDtypeStruct((NPAD, H), jnp.float32),
    )(x_pad, w, degp)


def _tc_mid_body(accb, degb, bb, wb, tableb):
    dinv = _dinv_block(degb[...])
    acc = accb[0] + accb[1]
    h = jnp.maximum(acc * dinv + bb[...], 0.0)
    t = jnp.dot(h, wb[...], preferred_element_type=jnp.float32)
    tableb[...] = t * dinv


def _tc_mid(accp, degp, b_row, w):
    return pl.pallas_call(
        _tc_mid_body,
        grid=(_GRID,),
        in_specs=[
            pl.BlockSpec((NC, _BM, H), lambda i: (0, i, 0)),
            pl.BlockSpec((NC, _BM, H), lambda i: (0, i, 0)),
            pl.BlockSpec((1, H), lambda i: (0, 0)),
            pl.BlockSpec((H, H), lambda i: (0, 0)),
        ],
        out_specs=pl.BlockSpec((_BM, H), lambda i: (i, 0)),
        out_shape=jax.ShapeDtypeStruct((NPAD, H), jnp.float32),
    )(accp, degp, b_row, w)


def _tc_final_body(accb, degb, bb, wlb, blb, batchb, y_out, gm_out,
                   sums_s, cnts_s):
    i = pl.program_id(0)

    @pl.when(i == 0)
    def _():
        sums_s[...] = jnp.zeros_like(sums_s)
        cnts_s[...] = jnp.zeros_like(cnts_s)

    dinv = _dinv_block(degb[...])
    acc = accb[0] + accb[1]
    h = jnp.maximum(acc * dinv + bb[...], 0.0)
    oh = (batchb[...] == lax.broadcasted_iota(jnp.int32, (_BM, G), 1))
    oh = oh.astype(jnp.float32)
    sums_s[...] += lax.dot_general(
        oh, h, (((0,), (0,)), ((), ())), preferred_element_type=jnp.float32)
    cnts_s[...] += lax.dot_general(
        oh, jnp.ones((_BM, 1), jnp.float32), (((0,), (0,)), ((), ())),
        preferred_element_type=jnp.float32)

    @pl.when(i == pl.num_programs(0) - 1)
    def _():
        gm = sums_s[...] / jnp.maximum(cnts_s[...], 1.0)
        gm_out[...] = gm
        y_out[...] = jnp.dot(gm, wlb[...],
                             preferred_element_type=jnp.float32) + blb[...]


def _tc_final(accp, degp, b_row, wl, bl_row, batch2d):
    return pl.pallas_call(
        _tc_final_body,
        grid=(_GRID,),
        in_specs=[
            pl.BlockSpec((NC, _BM, H), lambda i: (0, i, 0)),
            pl.BlockSpec((NC, _BM, H), lambda i: (0, i, 0)),
            pl.BlockSpec((1, H), lambda i: (0, 0)),
            pl.BlockSpec((H, C), lambda i: (0, 0)),
            pl.BlockSpec((1, C), lambda i: (0, 0)),
            pl.BlockSpec((_BM, 1), lambda i: (i, 0)),
        ],
        out_specs=[
            pl.BlockSpec((G, C), lambda i: (0, 0)),
            pl.BlockSpec((G, H), lambda i: (0, 0)),
        ],
        out_shape=[
            jax.ShapeDtypeStruct((G, C), jnp.float32),
            jax.ShapeDtypeStruct((G, H), jnp.float32),
        ],
        scratch_shapes=[
            pltpu.VMEM((G, H), jnp.float32),
            pltpu.VMEM((G, 1), jnp.float32),
        ],
    )(accp, degp, b_row, wl, bl_row, batch2d)


def kernel(x, edge_index, batch, W0, b0, W1, b1, W2, b2, Wl, bl):
    x_pad = jnp.pad(x, ((0, NPAD - N), (0, 0)))
    loops = jnp.arange(N, dtype=jnp.int32)
    padv = jnp.full((NGRP_PAD * EG - E_ALL,), N, jnp.int32)
    src_all = jnp.concatenate([edge_index[0], loops, padv])
    dst_all = jnp.concatenate([edge_index[1], loops, padv])
    pk_blk = (src_all | (dst_all << 16)).reshape(NGRP_PAD, EG)
    batch2d = jnp.pad(batch, (0, NPAD - N), constant_values=G).reshape(NPAD, 1)
    b0r = b0.reshape(1, H)
    b1r = b1.reshape(1, H)
    b2r = b2.reshape(1, H)
    blr = bl.reshape(1, C)

    deg_kernel, prop_kernel = _sc_kernels()
    degp = deg_kernel(pk_blk)

    table = _tc_first(x_pad, W0, degp)
    accp = prop_kernel(table, pk_blk)
    table = _tc_mid(accp, degp, b0r, W1)
    accp = prop_kernel(table, pk_blk)
    table = _tc_mid(accp, degp, b1r, W2)
    accp = prop_kernel(table, pk_blk)
    y, gm = _tc_final(accp, degp, b2r, Wl, blr, batch2d)
    return (y, gm)


# speedup vs baseline: 1.7220x; 1.7220x over previous
import functools

import jax
import jax.numpy as jnp
from jax import lax
from jax.experimental import pallas as pl
from jax.experimental.pallas import tpu as pltpu
from jax.experimental.pallas import tpu_sc as plsc

N = 10000
NPAD = 10240
E = 320000
D = 128
H = 128
C = 64
G = 128

NC = 2
NS = 16
NW = NC * NS
EG = 128
E_ALL = E + N
G0 = 120
G1 = 48
NGRP = NS * (G0 + G1)
NGRP_PAD = 2816
E_PAD = NGRP * EG
ROWS_PER_SUB = NPAD // NS


def _zero_vmem_rows(buf, nrows, width):
    z = jnp.zeros((16,), jnp.float32)

    def body(i, _):
        for j in range(width // 16):
            buf[i, pl.ds(j * 16, 16)] = z
        return 0

    lax.fori_loop(0, nrows, body, 0)


def _fill_vmem_rows(buf, nrows, width, value):
    v = jnp.full((16,), value, jnp.float32)

    def body(i, _):
        for j in range(width // 16):
            buf[i, pl.ds(j * 16, 16)] = v
        return 0

    lax.fori_loop(0, nrows, body, 0)


def _copy_rows_to_shared(buf, acc_sh, base):
    full, rem = divmod(ROWS_PER_SUB, EG)
    for g in range(full):
        pltpu.sync_copy(buf, acc_sh.at[pl.ds(base + g * EG, EG)])
    if rem:
        pltpu.sync_copy(buf.at[pl.ds(0, rem)],
                        acc_sh.at[pl.ds(base + full * EG, rem)])


def _unpack_group(pk_v, j, sbuf, dbuf):
    for k in range(EG // 16):
        v = pk_v[j, pl.ds(k * 16, 16)]
        if sbuf is not None:
            sbuf[pl.ds(k * 16, 16)] = v & 0xFFFF
        dbuf[pl.ds(k * 16, 16)] = v >> 16


DGPT = NGRP_PAD // NW


def _deg_body(pk_hbm, out_hbm, pk_v, ones_v, dbuf, acc_sh):
    c = lax.axis_index("c")
    s = lax.axis_index("s")
    wid = s * NC + c

    _zero_vmem_rows(ones_v, EG, H)
    _copy_rows_to_shared(ones_v, acc_sh, s * ROWS_PER_SUB)
    _fill_vmem_rows(ones_v, EG, H, 1.0)
    plsc.subcore_barrier()

    pltpu.sync_copy(pk_hbm.at[pl.ds(wid * DGPT, DGPT)], pk_v)

    def body(j, _):
        _unpack_group(pk_v, j, None, dbuf)
        pltpu.sync_copy(ones_v, acc_sh.at[dbuf], add=True)
        return 0

    lax.fori_loop(0, DGPT, body, 0)
    plsc.subcore_barrier()

    pltpu.sync_copy(
        acc_sh.at[pl.ds(s * ROWS_PER_SUB, ROWS_PER_SUB)],
        out_hbm.at[c, pl.ds(s * ROWS_PER_SUB, ROWS_PER_SUB)],
    )


def _pipelined_groups(table_hbm, pk_v, sa, da, sb, db, rows0, rows1,
                      acc_sh, semA, semB, ngroups):
    _unpack_group(pk_v, 0, sa, da)
    pltpu.async_copy(table_hbm.at[sa], rows0, semA)

    def body(jj, _):
        j0 = 2 * jj
        _unpack_group(pk_v, j0 + 1, sb, db)
        pltpu.async_copy(table_hbm.at[sb], rows1, semB)
        pltpu.make_async_copy(table_hbm.at[pl.ds(0, EG)], rows0, semA).wait()
        pltpu.sync_copy(rows0, acc_sh.at[da], add=True)

        _unpack_group(pk_v, jnp.minimum(j0 + 2, ngroups - 1), sa, da)
        pltpu.async_copy(table_hbm.at[sa], rows0, semA)

        pltpu.make_async_copy(table_hbm.at[pl.ds(0, EG)], rows1, semB).wait()
        pltpu.sync_copy(rows1, acc_sh.at[db], add=True)
        return 0

    lax.fori_loop(0, ngroups // 2, body, 0)
    pltpu.make_async_copy(table_hbm.at[pl.ds(0, EG)], rows0, semA).wait()


def _prop_body(table_hbm, pk_hbm, out_hbm, pk_v, sa, da, sb, db, rows0,
               rows1, acc_sh, semA, semB):
    c = lax.axis_index("c")
    s = lax.axis_index("s")

    _zero_vmem_rows(rows0, EG, H)
    _copy_rows_to_shared(rows0, acc_sh, s * ROWS_PER_SUB)
    plsc.subcore_barrier()

    base = jnp.where(c == 0, s * G0, NS * G0 + s * G1)
    pltpu.sync_copy(pk_hbm.at[pl.ds(base, G0)], pk_v)

    ngroups = jnp.where(c == 0, G0, G1)
    _pipelined_groups(table_hbm, pk_v, sa, da, sb, db, rows0, rows1,
                      acc_sh, semA, semB, ngroups)
    plsc.subcore_barrier()

    pltpu.sync_copy(
        acc_sh.at[pl.ds(s * ROWS_PER_SUB, ROWS_PER_SUB)],
        out_hbm.at[c, pl.ds(s * ROWS_PER_SUB, ROWS_PER_SUB)],
    )


@functools.cache
def _sc_kernels():
    mesh = plsc.VectorSubcoreMesh(core_axis_name="c", subcore_axis_name="s")
    deg = pl.kernel(
        _deg_body,
        out_type=jax.ShapeDtypeStruct((NC, NPAD, H), jnp.float32),
        mesh=mesh,
        scratch_types=[
            pltpu.VMEM((DGPT, EG), jnp.int32),
            pltpu.VMEM((EG, H), jnp.float32),
            pltpu.VMEM((EG,), jnp.int32),
            pltpu.VMEM_SHARED((NPAD, H), jnp.float32),
        ],
    )
    prop = pl.kernel(
        _prop_body,
        out_type=jax.ShapeDtypeStruct((NC, NPAD, H), jnp.float32),
        mesh=mesh,
        scratch_types=[
            pltpu.VMEM((G0, EG), jnp.int32),
            pltpu.VMEM((EG,), jnp.int32),
            pltpu.VMEM((EG,), jnp.int32),
            pltpu.VMEM((EG,), jnp.int32),
            pltpu.VMEM((EG,), jnp.int32),
            pltpu.VMEM((EG, H), jnp.float32),
            pltpu.VMEM((EG, H), jnp.float32),
            pltpu.VMEM_SHARED((NPAD, H), jnp.float32),
            pltpu.SemaphoreType.DMA,
            pltpu.SemaphoreType.DMA,
        ],
    )
    return deg, prop


_BM = 1024
_GRID = NPAD // _BM


def _dinv_block(degb):
    deg = degb[0, :, 0:1] + degb[1, :, 0:1]
    return lax.rsqrt(jnp.maximum(deg, 1.0))


def _tc_first_body(xb, wb, degb, tableb):
    t = jnp.dot(xb[...], wb[...], preferred_element_type=jnp.float32)
    tableb[...] = t * _dinv_block(degb[...])


def _tc_first(x_pad, w, degp):
    return pl.pallas_call(
        _tc_first_body,
        grid=(_GRID,),
        in_specs=[
            pl.BlockSpec((_BM, D), lambda i: (i, 0)),
            pl.BlockSpec((D, H), lambda i: (0, 0)),
            pl.BlockSpec((NC, _BM, H), lambda i: (0, i, 0)),
        ],
        out_specs=pl.BlockSpec((_BM, H), lambda i: (i, 0)),
        out_shape=jax.ShapeDtypeStruct((NPAD, H), jnp.float32),
    )(x_pad, w, degp)


def _tc_mid_body(accb, degb, bb, wb, tableb):
    dinv = _dinv_block(degb[...])
    acc = accb[0] + accb[1]
    h = jnp.maximum(acc * dinv + bb[...], 0.0)
    t = jnp.dot(h, wb[...], preferred_element_type=jnp.float32)
    tableb[...] = t * dinv


def _tc_mid(accp, degp, b_row, w):
    return pl.pallas_call(
        _tc_mid_body,
        grid=(_GRID,),
        in_specs=[
            pl.BlockSpec((NC, _BM, H), lambda i: (0, i, 0)),
            pl.BlockSpec((NC, _BM, H), lambda i: (0, i, 0)),
            pl.BlockSpec((1, H), lambda i: (0, 0)),
            pl.BlockSpec((H, H), lambda i: (0, 0)),
        ],
        out_specs=pl.BlockSpec((_BM, H), lambda i: (i, 0)),
        out_shape=jax.ShapeDtypeStruct((NPAD, H), jnp.float32),
    )(accp, degp, b_row, w)


def _tc_final_body(accb, degb, bb, wlb, blb, batchb, y_out, gm_out,
                   sums_s, cnts_s):
    i = pl.program_id(0)

    @pl.when(i == 0)
    def _():
        sums_s[...] = jnp.zeros_like(sums_s)
        cnts_s[...] = jnp.zeros_like(cnts_s)

    dinv = _dinv_block(degb[...])
    acc = accb[0] + accb[1]
    h = jnp.maximum(acc * dinv + bb[...], 0.0)
    oh = (batchb[...] == lax.broadcasted_iota(jnp.int32, (_BM, G), 1))
    oh = oh.astype(jnp.float32)
    sums_s[...] += lax.dot_general(
        oh, h, (((0,), (0,)), ((), ())), preferred_element_type=jnp.float32)
    cnts_s[...] += lax.dot_general(
        oh, jnp.ones((_BM, 1), jnp.float32), (((0,), (0,)), ((), ())),
        preferred_element_type=jnp.float32)

    @pl.when(i == pl.num_programs(0) - 1)
    def _():
        gm = sums_s[...] / jnp.maximum(cnts_s[...], 1.0)
        gm_out[...] = gm
        y_out[...] = jnp.dot(gm, wlb[...],
                             preferred_element_type=jnp.float32) + blb[...]


def _tc_final(accp, degp, b_row, wl, bl_row, batch2d):
    return pl.pallas_call(
        _tc_final_body,
        grid=(_GRID,),
        in_specs=[
            pl.BlockSpec((NC, _BM, H), lambda i: (0, i, 0)),
            pl.BlockSpec((NC, _BM, H), lambda i: (0, i, 0)),
            pl.BlockSpec((1, H), lambda i: (0, 0)),
            pl.BlockSpec((H, C), lambda i: (0, 0)),
            pl.BlockSpec((1, C), lambda i: (0, 0)),
            pl.BlockSpec((_BM, 1), lambda i: (i, 0)),
        ],
        out_specs=[
            pl.BlockSpec((G, C), lambda i: (0, 0)),
            pl.BlockSpec((G, H), lambda i: (0, 0)),
        ],
        out_shape=[
            jax.ShapeDtypeStruct((G, C), jnp.float32),
            jax.ShapeDtypeStruct((G, H), jnp.float32),
        ],
        scratch_shapes=[
            pltpu.VMEM((G, H), jnp.float32),
            pltpu.VMEM((G, 1), jnp.float32),
        ],
    )(accp, degp, b_row, wl, bl_row, batch2d)


def kernel(x, edge_index, batch, W0, b0, W1, b1, W2, b2, Wl, bl):
    x_pad = jnp.pad(x, ((0, NPAD - N), (0, 0)))
    loops = jnp.arange(N, dtype=jnp.int32)
    padv = jnp.full((NGRP_PAD * EG - E_ALL,), N, jnp.int32)
    src_all = jnp.concatenate([edge_index[0], loops, padv])
    dst_all = jnp.concatenate([edge_index[1], loops, padv])
    pk_blk = (src_all | (dst_all << 16)).reshape(NGRP_PAD, EG)
    batch2d = jnp.pad(batch, (0, NPAD - N), constant_values=G).reshape(NPAD, 1)
    b0r = b0.reshape(1, H)
    b1r = b1.reshape(1, H)
    b2r = b2.reshape(1, H)
    blr = bl.reshape(1, C)

    deg_kernel, prop_kernel = _sc_kernels()
    degp = deg_kernel(pk_blk)

    table = _tc_first(x_pad, W0, degp)
    accp = prop_kernel(table, pk_blk)
    table = _tc_mid(accp, degp, b0r, W1)
    accp = prop_kernel(table, pk_blk)
    table = _tc_mid(accp, degp, b1r, W2)
    accp = prop_kernel(table, pk_blk)
    y, gm = _tc_final(accp, degp, b2r, Wl, blr, batch2d)
    return (y, gm)


# speedup vs baseline: 1.7225x; 1.0003x over previous
import functools

import jax
import jax.numpy as jnp
from jax import lax
from jax.experimental import pallas as pl
from jax.experimental.pallas import tpu as pltpu
from jax.experimental.pallas import tpu_sc as plsc

N = 10000
NPAD = 10240
E = 320000
D = 128
H = 128
C = 64
G = 128

NC = 2
NS = 16
NW = NC * NS
EG = 128
E_ALL = E + N
G0 = 120
G1 = 48
NGRP = NS * (G0 + G1)
NGRP_PAD = 2816
E_PAD = NGRP * EG
ROWS_PER_SUB = NPAD // NS


def _zero_vmem_rows(buf, nrows, width):
    z = jnp.zeros((16,), jnp.float32)

    def body(i, _):
        for j in range(width // 16):
            buf[i, pl.ds(j * 16, 16)] = z
        return 0

    lax.fori_loop(0, nrows, body, 0)


def _fill_vmem_rows(buf, nrows, width, value):
    v = jnp.full((16,), value, jnp.float32)

    def body(i, _):
        for j in range(width // 16):
            buf[i, pl.ds(j * 16, 16)] = v
        return 0

    lax.fori_loop(0, nrows, body, 0)


def _copy_rows_to_shared(buf, acc_sh, base):
    full, rem = divmod(ROWS_PER_SUB, EG)
    for g in range(full):
        pltpu.sync_copy(buf, acc_sh.at[pl.ds(base + g * EG, EG)])
    if rem:
        pltpu.sync_copy(buf.at[pl.ds(0, rem)],
                        acc_sh.at[pl.ds(base + full * EG, rem)])


def _unpack_group(pk_v, j, sbuf, dbuf):
    for k in range(EG // 16):
        v = pk_v[j, pl.ds(k * 16, 16)]
        if sbuf is not None:
            sbuf[pl.ds(k * 16, 16)] = v & 0xFFFF
        dbuf[pl.ds(k * 16, 16)] = v >> 16


DGPT = NGRP_PAD // NW


def _deg_body(pk_hbm, out_hbm, pk_v, ones_v, dbuf, acc_sh):
    c = lax.axis_index("c")
    s = lax.axis_index("s")
    wid = s * NC + c

    _zero_vmem_rows(ones_v, EG, H)
    _copy_rows_to_shared(ones_v, acc_sh, s * ROWS_PER_SUB)
    _fill_vmem_rows(ones_v, EG, H, 1.0)
    plsc.subcore_barrier()

    pltpu.sync_copy(pk_hbm.at[pl.ds(wid * DGPT, DGPT)], pk_v)

    def body(j, _):
        _unpack_group(pk_v, j, None, dbuf)
        pltpu.sync_copy(ones_v, acc_sh.at[dbuf], add=True)
        return 0

    lax.fori_loop(0, DGPT, body, 0)
    plsc.subcore_barrier()

    pltpu.sync_copy(
        acc_sh.at[pl.ds(s * ROWS_PER_SUB, ROWS_PER_SUB)],
        out_hbm.at[c, pl.ds(s * ROWS_PER_SUB, ROWS_PER_SUB)],
    )


def _pipelined_groups(table_hbm, pk_v, sa, da, sb, db, rows0, rows1,
                      acc_sh, semA, semB, nhalf):
    _unpack_group(pk_v, 0, sa, da)
    pltpu.async_copy(table_hbm.at[sa], rows0, semA)

    def body(jj, _):
        @pl.when(jj < nhalf)
        def _():
            j0 = 2 * jj
            _unpack_group(pk_v, j0 + 1, sb, db)
            pltpu.async_copy(table_hbm.at[sb], rows1, semB)
            pltpu.make_async_copy(
                table_hbm.at[pl.ds(0, EG)], rows0, semA).wait()
            pltpu.sync_copy(rows0, acc_sh.at[da], add=True)

            _unpack_group(pk_v, jnp.minimum(j0 + 2, 2 * nhalf - 1), sa, da)
            pltpu.async_copy(table_hbm.at[sa], rows0, semA)

            pltpu.make_async_copy(
                table_hbm.at[pl.ds(0, EG)], rows1, semB).wait()
            pltpu.sync_copy(rows1, acc_sh.at[db], add=True)
        return 0

    lax.fori_loop(0, G0 // 2, body, 0)
    pltpu.make_async_copy(table_hbm.at[pl.ds(0, EG)], rows0, semA).wait()


def _prop_body(table_hbm, pk_hbm, out_hbm, pk_v, sa, da, sb, db, rows0,
               rows1, acc_sh, semA, semB):
    c = lax.axis_index("c")
    s = lax.axis_index("s")

    _zero_vmem_rows(rows0, EG, H)
    _copy_rows_to_shared(rows0, acc_sh, s * ROWS_PER_SUB)
    plsc.subcore_barrier()

    base = jnp.where(c == 0, s * G0, NS * G0 + s * G1)
    pltpu.sync_copy(pk_hbm.at[pl.ds(base, G0)], pk_v)

    nhalf = jnp.where(c == 0, G0 // 2, G1 // 2)
    _pipelined_groups(table_hbm, pk_v, sa, da, sb, db, rows0, rows1,
                      acc_sh, semA, semB, nhalf)
    plsc.subcore_barrier()

    pltpu.sync_copy(
        acc_sh.at[pl.ds(s * ROWS_PER_SUB, ROWS_PER_SUB)],
        out_hbm.at[c, pl.ds(s * ROWS_PER_SUB, ROWS_PER_SUB)],
    )


@functools.cache
def _sc_kernels():
    mesh = plsc.VectorSubcoreMesh(core_axis_name="c", subcore_axis_name="s")
    deg = pl.kernel(
        _deg_body,
        out_type=jax.ShapeDtypeStruct((NC, NPAD, H), jnp.float32),
        mesh=mesh,
        scratch_types=[
            pltpu.VMEM((DGPT, EG), jnp.int32),
            pltpu.VMEM((EG, H), jnp.float32),
            pltpu.VMEM((EG,), jnp.int32),
            pltpu.VMEM_SHARED((NPAD, H), jnp.float32),
        ],
    )
    prop = pl.kernel(
        _prop_body,
        out_type=jax.ShapeDtypeStruct((NC, NPAD, H), jnp.float32),
        mesh=mesh,
        scratch_types=[
            pltpu.VMEM((G0, EG), jnp.int32),
            pltpu.VMEM((EG,), jnp.int32),
            pltpu.VMEM((EG,), jnp.int32),
            pltpu.VMEM((EG,), jnp.int32),
            pltpu.VMEM((EG,), jnp.int32),
            pltpu.VMEM((EG, H), jnp.float32),
            pltpu.VMEM((EG, H), jnp.float32),
            pltpu.VMEM_SHARED((NPAD, H), jnp.float32),
            pltpu.SemaphoreType.DMA,
            pltpu.SemaphoreType.DMA,
        ],
    )
    return deg, prop


_BM = 1024
_GRID = NPAD // _BM


def _dinv_block(degb):
    deg = degb[0, :, 0:1] + degb[1, :, 0:1]
    return lax.rsqrt(jnp.maximum(deg, 1.0))


def _tc_first_body(xb, wb, degb, tableb):
    t = jnp.dot(xb[...], wb[...], preferred_element_type=jnp.float32)
    tableb[...] = t * _dinv_block(degb[...])


def _tc_first(x_pad, w, degp):
    return pl.pallas_call(
        _tc_first_body,
        grid=(_GRID,),
        in_specs=[
            pl.BlockSpec((_BM, D), lambda i: (i, 0)),
            pl.BlockSpec((D, H), lambda i: (0, 0)),
            pl.BlockSpec((NC, _BM, H), lambda i: (0, i, 0)),
        ],
        out_specs=pl.BlockSpec((_BM, H), lambda i: (i, 0)),
        out_shape=jax.ShapeDtypeStruct((NPAD, H), jnp.float32),
    )(x_pad, w, degp)


def _tc_mid_body(accb, degb, bb, wb, tableb):
    dinv = _dinv_block(degb[...])
    acc = accb[0] + accb[1]
    h = jnp.maximum(acc * dinv + bb[...], 0.0)
    t = jnp.dot(h, wb[...], preferred_element_type=jnp.float32)
    tableb[...] = t * dinv


def _tc_mid(accp, degp, b_row, w):
    return pl.pallas_call(
        _tc_mid_body,
        grid=(_GRID,),
        in_specs=[
            pl.BlockSpec((NC, _BM, H), lambda i: (0, i, 0)),
            pl.BlockSpec((NC, _BM, H), lambda i: (0, i, 0)),
            pl.BlockSpec((1, H), lambda i: (0, 0)),
            pl.BlockSpec((H, H), lambda i: (0, 0)),
        ],
        out_specs=pl.BlockSpec((_BM, H), lambda i: (i, 0)),
        out_shape=jax.ShapeDtypeStruct((NPAD, H), jnp.float32),
    )(accp, degp, b_row, w)


def _tc_final_body(accb, degb, bb, wlb, blb, batchb, y_out, gm_out,
                   sums_s, cnts_s):
    i = pl.program_id(0)

    @pl.when(i == 0)
    def _():
        sums_s[...] = jnp.zeros_like(sums_s)
        cnts_s[...] = jnp.zeros_like(cnts_s)

    dinv = _dinv_block(degb[...])
    acc = accb[0] + accb[1]
    h = jnp.maximum(acc * dinv + bb[...], 0.0)
    oh = (batchb[...] == lax.broadcasted_iota(jnp.int32, (_BM, G), 1))
    oh = oh.astype(jnp.float32)
    sums_s[...] += lax.dot_general(
        oh, h, (((0,), (0,)), ((), ())), preferred_element_type=jnp.float32)
    cnts_s[...] += lax.dot_general(
        oh, jnp.ones((_BM, 1), jnp.float32), (((0,), (0,)), ((), ())),
        preferred_element_type=jnp.float32)

    @pl.when(i == pl.num_programs(0) - 1)
    def _():
        gm = sums_s[...] / jnp.maximum(cnts_s[...], 1.0)
        gm_out[...] = gm
        y_out[...] = jnp.dot(gm, wlb[...],
                             preferred_element_type=jnp.float32) + blb[...]


def _tc_final(accp, degp, b_row, wl, bl_row, batch2d):
    return pl.pallas_call(
        _tc_final_body,
        grid=(_GRID,),
        in_specs=[
            pl.BlockSpec((NC, _BM, H), lambda i: (0, i, 0)),
            pl.BlockSpec((NC, _BM, H), lambda i: (0, i, 0)),
            pl.BlockSpec((1, H), lambda i: (0, 0)),
            pl.BlockSpec((H, C), lambda i: (0, 0)),
            pl.BlockSpec((1, C), lambda i: (0, 0)),
            pl.BlockSpec((_BM, 1), lambda i: (i, 0)),
        ],
        out_specs=[
            pl.BlockSpec((G, C), lambda i: (0, 0)),
            pl.BlockSpec((G, H), lambda i: (0, 0)),
        ],
        out_shape=[
            jax.ShapeDtypeStruct((G, C), jnp.float32),
            jax.ShapeDtypeStruct((G, H), jnp.float32),
        ],
        scratch_shapes=[
            pltpu.VMEM((G, H), jnp.float32),
            pltpu.VMEM((G, 1), jnp.float32),
        ],
    )(accp, degp, b_row, wl, bl_row, batch2d)


def kernel(x, edge_index, batch, W0, b0, W1, b1, W2, b2, Wl, bl):
    x_pad = jnp.pad(x, ((0, NPAD - N), (0, 0)))
    loops = jnp.arange(N, dtype=jnp.int32)
    padv = jnp.full((NGRP_PAD * EG - E_ALL,), N, jnp.int32)
    src_all = jnp.concatenate([edge_index[0], loops, padv])
    dst_all = jnp.concatenate([edge_index[1], loops, padv])
    pk_blk = (src_all | (dst_all << 16)).reshape(NGRP_PAD, EG)
    batch2d = jnp.pad(batch, (0, NPAD - N), constant_values=G).reshape(NPAD, 1)
    b0r = b0.reshape(1, H)
    b1r = b1.reshape(1, H)
    b2r = b2.reshape(1, H)
    blr = bl.reshape(1, C)

    deg_kernel, prop_kernel = _sc_kernels()
    degp = deg_kernel(pk_blk)

    table = _tc_first(x_pad, W0, degp)
    accp = prop_kernel(table, pk_blk)
    table = _tc_mid(accp, degp, b0r, W1)
    accp = prop_kernel(table, pk_blk)
    table = _tc_mid(accp, degp, b1r, W2)
    accp = prop_kernel(table, pk_blk)
    y, gm = _tc_final(accp, degp, b2r, Wl, blr, batch2d)
    return (y, gm)


# speedup vs baseline: 7.9409x; 4.6100x over previous
import functools

import jax
import jax.numpy as jnp
from jax import lax
from jax.experimental import pallas as pl
from jax.experimental.pallas import tpu as pltpu
from jax.experimental.pallas import tpu_sc as plsc

N = 10000
NPAD = 10240
E = 320000
D = 128
H = 128
C = 64
G = 128

NC = 2
NS = 16
NW = NC * NS
EG = 128
E_ALL = E + N
GPT = 82
E_PAD = NW * EG * GPT
ROWS_PER_SUB = NPAD // NS


def _zero_vmem_rows(buf, nrows, width):
    z = jnp.zeros((16,), jnp.float32)

    def body(i, _):
        for j in range(width // 16):
            buf[i, pl.ds(j * 16, 16)] = z
        return 0

    lax.fori_loop(0, nrows, body, 0)


def _fill_vmem_rows(buf, nrows, width, value):
    v = jnp.full((16,), value, jnp.float32)

    def body(i, _):
        for j in range(width // 16):
            buf[i, pl.ds(j * 16, 16)] = v
        return 0

    lax.fori_loop(0, nrows, body, 0)


def _copy_rows_to_shared(buf, acc_sh, base):
    full, rem = divmod(ROWS_PER_SUB, EG)
    for g in range(full):
        pltpu.sync_copy(buf, acc_sh.at[pl.ds(base + g * EG, EG)])
    if rem:
        pltpu.sync_copy(buf.at[pl.ds(0, rem)],
                        acc_sh.at[pl.ds(base + full * EG, rem)])


def _unpack_group(pk_v, j, sbuf, dbuf):
    for k in range(EG // 16):
        v = pk_v[j, pl.ds(k * 16, 16)]
        if sbuf is not None:
            sbuf[pl.ds(k * 16, 16)] = v & 0xFFFF
        dbuf[pl.ds(k * 16, 16)] = v >> 16


def _deg_body(pk_hbm, out_hbm, pk_v, ones_v, dbuf, acc_sh):
    c = lax.axis_index("c")
    s = lax.axis_index("s")
    wid = s * NC + c

    _zero_vmem_rows(ones_v, EG, H)
    _copy_rows_to_shared(ones_v, acc_sh, s * ROWS_PER_SUB)
    _fill_vmem_rows(ones_v, EG, H, 1.0)
    plsc.subcore_barrier()

    pltpu.sync_copy(pk_hbm.at[wid], pk_v)

    def body(j, _):
        _unpack_group(pk_v, j, None, dbuf)
        pltpu.sync_copy(ones_v, acc_sh.at[dbuf], add=True)
        return 0

    lax.fori_loop(0, GPT, body, 0)
    plsc.subcore_barrier()

    pltpu.sync_copy(
        acc_sh.at[pl.ds(s * ROWS_PER_SUB, ROWS_PER_SUB)],
        out_hbm.at[c, pl.ds(s * ROWS_PER_SUB, ROWS_PER_SUB)],
    )


def _prop_body(table_hbm, pk_hbm, out_hbm, pk_v, sa, da, sb, db, rows0,
               rows1, acc_sh, semA, semB):
    c = lax.axis_index("c")
    s = lax.axis_index("s")
    wid = s * NC + c

    _zero_vmem_rows(rows0, EG, H)
    _copy_rows_to_shared(rows0, acc_sh, s * ROWS_PER_SUB)
    plsc.subcore_barrier()

    pltpu.sync_copy(pk_hbm.at[wid], pk_v)

    _unpack_group(pk_v, 0, sa, da)
    pltpu.async_copy(table_hbm.at[sa], rows0, semA)

    def body(jj, _):
        j0 = 2 * jj
        _unpack_group(pk_v, j0 + 1, sb, db)
        pltpu.async_copy(table_hbm.at[sb], rows1, semB)
        pltpu.make_async_copy(table_hbm.at[pl.ds(0, EG)], rows0, semA).wait()
        pltpu.sync_copy(rows0, acc_sh.at[da], add=True)

        _unpack_group(pk_v, jnp.minimum(j0 + 2, GPT - 1), sa, da)
        pltpu.async_copy(table_hbm.at[sa], rows0, semA)

        pltpu.make_async_copy(table_hbm.at[pl.ds(0, EG)], rows1, semB).wait()
        pltpu.sync_copy(rows1, acc_sh.at[db], add=True)
        return 0

    lax.fori_loop(0, GPT // 2, body, 0)
    pltpu.make_async_copy(table_hbm.at[pl.ds(0, EG)], rows0, semA).wait()
    plsc.subcore_barrier()

    pltpu.sync_copy(
        acc_sh.at[pl.ds(s * ROWS_PER_SUB, ROWS_PER_SUB)],
        out_hbm.at[c, pl.ds(s * ROWS_PER_SUB, ROWS_PER_SUB)],
    )


@functools.cache
def _sc_kernels():
    mesh = plsc.VectorSubcoreMesh(core_axis_name="c", subcore_axis_name="s")
    deg = pl.kernel(
        _deg_body,
        out_type=jax.ShapeDtypeStruct((NC, NPAD, H), jnp.float32),
        mesh=mesh,
        scratch_types=[
            pltpu.VMEM((GPT, EG), jnp.int32),
            pltpu.VMEM((EG, H), jnp.float32),
            pltpu.VMEM((EG,), jnp.int32),
            pltpu.VMEM_SHARED((NPAD, H), jnp.float32),
        ],
    )
    prop = pl.kernel(
        _prop_body,
        out_type=jax.ShapeDtypeStruct((NC, NPAD, H), jnp.float32),
        mesh=mesh,
        scratch_types=[
            pltpu.VMEM((GPT, EG), jnp.int32),
            pltpu.VMEM((EG,), jnp.int32),
            pltpu.VMEM((EG,), jnp.int32),
            pltpu.VMEM((EG,), jnp.int32),
            pltpu.VMEM((EG,), jnp.int32),
            pltpu.VMEM((EG, H), jnp.float32),
            pltpu.VMEM((EG, H), jnp.float32),
            pltpu.VMEM_SHARED((NPAD, H), jnp.float32),
            pltpu.SemaphoreType.DMA,
            pltpu.SemaphoreType.DMA,
        ],
    )
    return deg, prop


_BM = 1024
_GRID = NPAD // _BM


def _dinv_block(degb):
    deg = degb[0, :, 0:1] + degb[1, :, 0:1]
    return lax.rsqrt(jnp.maximum(deg, 1.0))


def _tc_first_body(xb, wb, degb, tableb):
    t = jnp.dot(xb[...], wb[...], preferred_element_type=jnp.float32)
    tableb[...] = t * _dinv_block(degb[...])


def _tc_first(x_pad, w, degp):
    return pl.pallas_call(
        _tc_first_body,
        grid=(_GRID,),
        in_specs=[
            pl.BlockSpec((_BM, D), lambda i: (i, 0)),
            pl.BlockSpec((D, H), lambda i: (0, 0)),
            pl.BlockSpec((NC, _BM, H), lambda i: (0, i, 0)),
        ],
        out_specs=pl.BlockSpec((_BM, H), lambda i: (i, 0)),
        out_shape=jax.ShapeDtypeStruct((NPAD, H), jnp.float32),
    )(x_pad, w, degp)


def _tc_mid_body(accb, degb, bb, wb, tableb):
    dinv = _dinv_block(degb[...])
    acc = accb[0] + accb[1]
    h = jnp.maximum(acc * dinv + bb[...], 0.0)
    t = jnp.dot(h, wb[...], preferred_element_type=jnp.float32)
    tableb[...] = t * dinv


def _tc_mid(accp, degp, b_row, w):
    return pl.pallas_call(
        _tc_mid_body,
        grid=(_GRID,),
        in_specs=[
            pl.BlockSpec((NC, _BM, H), lambda i: (0, i, 0)),
            pl.BlockSpec((NC, _BM, H), lambda i: (0, i, 0)),
            pl.BlockSpec((1, H), lambda i: (0, 0)),
            pl.BlockSpec((H, H), lambda i: (0, 0)),
        ],
        out_specs=pl.BlockSpec((_BM, H), lambda i: (i, 0)),
        out_shape=jax.ShapeDtypeStruct((NPAD, H), jnp.float32),
    )(accp, degp, b_row, w)


def _tc_final_body(accb, degb, bb, wlb, blb, batchb, y_out, gm_out,
                   sums_s, cnts_s):
    i = pl.program_id(0)

    @pl.when(i == 0)
    def _():
        sums_s[...] = jnp.zeros_like(sums_s)
        cnts_s[...] = jnp.zeros_like(cnts_s)

    dinv = _dinv_block(degb[...])
    acc = accb[0] + accb[1]
    h = jnp.maximum(acc * dinv + bb[...], 0.0)
    oh = (batchb[...] == lax.broadcasted_iota(jnp.int32, (_BM, G), 1))
    oh = oh.astype(jnp.float32)
    sums_s[...] += lax.dot_general(
        oh, h, (((0,), (0,)), ((), ())), preferred_element_type=jnp.float32)
    cnts_s[...] += lax.dot_general(
        oh, jnp.ones((_BM, 1), jnp.float32), (((0,), (0,)), ((), ())),
        preferred_element_type=jnp.float32)

    @pl.when(i == pl.num_programs(0) - 1)
    def _():
        gm = sums_s[...] / jnp.maximum(cnts_s[...], 1.0)
        gm_out[...] = gm
        y_out[...] = jnp.dot(gm, wlb[...],
                             preferred_element_type=jnp.float32) + blb[...]


def _tc_final(accp, degp, b_row, wl, bl_row, batch2d):
    return pl.pallas_call(
        _tc_final_body,
        grid=(_GRID,),
        in_specs=[
            pl.BlockSpec((NC, _BM, H), lambda i: (0, i, 0)),
            pl.BlockSpec((NC, _BM, H), lambda i: (0, i, 0)),
            pl.BlockSpec((1, H), lambda i: (0, 0)),
            pl.BlockSpec((H, C), lambda i: (0, 0)),
            pl.BlockSpec((1, C), lambda i: (0, 0)),
            pl.BlockSpec((_BM, 1), lambda i: (i, 0)),
        ],
        out_specs=[
            pl.BlockSpec((G, C), lambda i: (0, 0)),
            pl.BlockSpec((G, H), lambda i: (0, 0)),
        ],
        out_shape=[
            jax.ShapeDtypeStruct((G, C), jnp.float32),
            jax.ShapeDtypeStruct((G, H), jnp.float32),
        ],
        scratch_shapes=[
            pltpu.VMEM((G, H), jnp.float32),
            pltpu.VMEM((G, 1), jnp.float32),
        ],
    )(accp, degp, b_row, wl, bl_row, batch2d)


def kernel(x, edge_index, batch, W0, b0, W1, b1, W2, b2, Wl, bl):
    x_pad = jnp.pad(x, ((0, NPAD - N), (0, 0)))
    loops = jnp.arange(N, dtype=jnp.int32)
    padv = N + (jnp.arange(E_PAD - E_ALL, dtype=jnp.int32) % (NPAD - N))
    src_all = jnp.concatenate([edge_index[0], loops, padv])
    dst_all = jnp.concatenate([edge_index[1], loops, padv])
    pk_blk = (src_all | (dst_all << 16)).reshape(NW, GPT, EG)
    batch2d = jnp.pad(batch, (0, NPAD - N), constant_values=G).reshape(NPAD, 1)
    b0r = b0.reshape(1, H)
    b1r = b1.reshape(1, H)
    b2r = b2.reshape(1, H)
    blr = bl.reshape(1, C)

    deg_kernel, prop_kernel = _sc_kernels()
    degp = deg_kernel(pk_blk)

    table = _tc_first(x_pad, W0, degp)
    accp = prop_kernel(table, pk_blk)
    table = _tc_mid(accp, degp, b0r, W1)
    accp = prop_kernel(table, pk_blk)
    table = _tc_mid(accp, degp, b1r, W2)
    accp = prop_kernel(table, pk_blk)
    y, gm = _tc_final(accp, degp, b2r, Wl, blr, batch2d)
    return (y, gm)


# speedup vs baseline: 8.0554x; 1.0144x over previous
import functools

import jax
import jax.numpy as jnp
from jax import lax
from jax.experimental import pallas as pl
from jax.experimental.pallas import tpu as pltpu
from jax.experimental.pallas import tpu_sc as plsc

N = 10000
NPAD = 10240
E = 320000
D = 128
H = 128
C = 64
G = 128

NC = 2
NS = 16
NW = NC * NS
EG = 128
E_ALL = E + N
GPT = 82
E_PAD = NW * EG * GPT
ROWS_PER_SUB = NPAD // NS


def _zero_vmem_rows(buf, nrows, width):
    z = jnp.zeros((16,), jnp.float32)

    def body(i, _):
        for j in range(width // 16):
            buf[i, pl.ds(j * 16, 16)] = z
        return 0

    lax.fori_loop(0, nrows, body, 0)


def _fill_vmem_rows(buf, nrows, width, value):
    v = jnp.full((16,), value, jnp.float32)

    def body(i, _):
        for j in range(width // 16):
            buf[i, pl.ds(j * 16, 16)] = v
        return 0

    lax.fori_loop(0, nrows, body, 0)


def _copy_rows_to_shared(buf, acc_sh, base):
    full, rem = divmod(ROWS_PER_SUB, EG)
    for g in range(full):
        pltpu.sync_copy(buf, acc_sh.at[pl.ds(base + g * EG, EG)])
    if rem:
        pltpu.sync_copy(buf.at[pl.ds(0, rem)],
                        acc_sh.at[pl.ds(base + full * EG, rem)])


def _unpack_group(pk_v, j, sbuf, dbuf):
    for k in range(EG // 16):
        v = pk_v[j, pl.ds(k * 16, 16)]
        if sbuf is not None:
            sbuf[pl.ds(k * 16, 16)] = v & 0xFFFF
        dbuf[pl.ds(k * 16, 16)] = v >> 16


def _deg_body(pk_hbm, out_hbm, pk_v, ones_v, dbuf, acc_sh):
    c = lax.axis_index("c")
    s = lax.axis_index("s")
    wid = s * NC + c

    _zero_vmem_rows(ones_v, EG, H)
    _copy_rows_to_shared(ones_v, acc_sh, s * ROWS_PER_SUB)
    _fill_vmem_rows(ones_v, EG, H, 1.0)
    plsc.subcore_barrier()

    pltpu.sync_copy(pk_hbm.at[wid], pk_v)

    def body(j, _):
        _unpack_group(pk_v, j, None, dbuf)
        pltpu.sync_copy(ones_v, acc_sh.at[dbuf], add=True)
        return 0

    lax.fori_loop(0, GPT, body, 0)
    plsc.subcore_barrier()

    pltpu.sync_copy(
        acc_sh.at[pl.ds(s * ROWS_PER_SUB, ROWS_PER_SUB)],
        out_hbm.at[c, pl.ds(s * ROWS_PER_SUB, ROWS_PER_SUB)],
    )


def _prop_body(table_hbm, pk_hbm, out_hbm, pk_v, sa, da, sb, db, rows0,
               rows1, acc_sh, semA, semB):
    c = lax.axis_index("c")
    s = lax.axis_index("s")
    wid = s * NC + c

    _zero_vmem_rows(rows0, EG, H)
    _copy_rows_to_shared(rows0, acc_sh, s * ROWS_PER_SUB)
    plsc.subcore_barrier()

    pltpu.sync_copy(pk_hbm.at[wid], pk_v)

    _unpack_group(pk_v, 0, sa, da)
    pltpu.async_copy(table_hbm.at[sa], rows0, semA)

    def body(jj, _):
        j0 = 2 * jj
        _unpack_group(pk_v, j0 + 1, sb, db)
        pltpu.async_copy(table_hbm.at[sb], rows1, semB)
        pltpu.make_async_copy(table_hbm.at[pl.ds(0, EG)], rows0, semA).wait()
        pltpu.sync_copy(rows0, acc_sh.at[da], add=True)

        _unpack_group(pk_v, jnp.minimum(j0 + 2, GPT - 1), sa, da)
        pltpu.async_copy(table_hbm.at[sa], rows0, semA)

        pltpu.make_async_copy(table_hbm.at[pl.ds(0, EG)], rows1, semB).wait()
        pltpu.sync_copy(rows1, acc_sh.at[db], add=True)
        return 0

    lax.fori_loop(0, GPT // 2, body, 0)
    pltpu.make_async_copy(table_hbm.at[pl.ds(0, EG)], rows0, semA).wait()
    plsc.subcore_barrier()

    pltpu.sync_copy(
        acc_sh.at[pl.ds(s * ROWS_PER_SUB, ROWS_PER_SUB)],
        out_hbm.at[c, pl.ds(s * ROWS_PER_SUB, ROWS_PER_SUB)],
    )


@functools.cache
def _sc_kernels():
    mesh = plsc.VectorSubcoreMesh(core_axis_name="c", subcore_axis_name="s")
    deg = pl.kernel(
        _deg_body,
        out_type=jax.ShapeDtypeStruct((NC, NPAD, H), jnp.float32),
        mesh=mesh,
        scratch_types=[
            pltpu.VMEM((GPT, EG), jnp.int32),
            pltpu.VMEM((EG, H), jnp.float32),
            pltpu.VMEM((EG,), jnp.int32),
            pltpu.VMEM_SHARED((NPAD, H), jnp.float32),
        ],
    )
    prop = pl.kernel(
        _prop_body,
        out_type=jax.ShapeDtypeStruct((NC, NPAD, H), jnp.float32),
        mesh=mesh,
        scratch_types=[
            pltpu.VMEM((GPT, EG), jnp.int32),
            pltpu.VMEM((EG,), jnp.int32),
            pltpu.VMEM((EG,), jnp.int32),
            pltpu.VMEM((EG,), jnp.int32),
            pltpu.VMEM((EG,), jnp.int32),
            pltpu.VMEM((EG, H), jnp.float32),
            pltpu.VMEM((EG, H), jnp.float32),
            pltpu.VMEM_SHARED((NPAD, H), jnp.float32),
            pltpu.SemaphoreType.DMA,
            pltpu.SemaphoreType.DMA,
        ],
    )
    return deg, prop


_BM = 1024
_GRID = NPAD // _BM


def _dinv_block(degb):
    deg = degb[0, :, 0:1] + degb[1, :, 0:1]
    return lax.rsqrt(jnp.maximum(deg, 1.0))


def _tc_matmul_body(xb, wb, tb):
    tb[...] = jnp.dot(xb[...], wb[...], preferred_element_type=jnp.float32)


def _tc_matmul(x_pad, w):
    return pl.pallas_call(
        _tc_matmul_body,
        grid=(_GRID,),
        in_specs=[
            pl.BlockSpec((_BM, D), lambda i: (i, 0)),
            pl.BlockSpec((D, H), lambda i: (0, 0)),
        ],
        out_specs=pl.BlockSpec((_BM, H), lambda i: (i, 0)),
        out_shape=jax.ShapeDtypeStruct((NPAD, H), jnp.float32),
    )(x_pad, w)


def _tc_scale_body(tb, degb, tableb):
    tableb[...] = tb[...] * _dinv_block(degb[...])


def _tc_scale(t0, degp):
    return pl.pallas_call(
        _tc_scale_body,
        grid=(_GRID,),
        in_specs=[
            pl.BlockSpec((_BM, H), lambda i: (i, 0)),
            pl.BlockSpec((NC, _BM, H), lambda i: (0, i, 0)),
        ],
        out_specs=pl.BlockSpec((_BM, H), lambda i: (i, 0)),
        out_shape=jax.ShapeDtypeStruct((NPAD, H), jnp.float32),
    )(t0, degp)


def _tc_mid_body(accb, degb, bb, wb, tableb):
    dinv = _dinv_block(degb[...])
    acc = accb[0] + accb[1]
    h = jnp.maximum(acc * dinv + bb[...], 0.0)
    t = jnp.dot(h, wb[...], preferred_element_type=jnp.float32)
    tableb[...] = t * dinv


def _tc_mid(accp, degp, b_row, w):
    return pl.pallas_call(
        _tc_mid_body,
        grid=(_GRID,),
        in_specs=[
            pl.BlockSpec((NC, _BM, H), lambda i: (0, i, 0)),
            pl.BlockSpec((NC, _BM, H), lambda i: (0, i, 0)),
            pl.BlockSpec((1, H), lambda i: (0, 0)),
            pl.BlockSpec((H, H), lambda i: (0, 0)),
        ],
        out_specs=pl.BlockSpec((_BM, H), lambda i: (i, 0)),
        out_shape=jax.ShapeDtypeStruct((NPAD, H), jnp.float32),
    )(accp, degp, b_row, w)


def _tc_final_body(accb, degb, bb, wlb, blb, batchb, y_out, gm_out,
                   sums_s, cnts_s):
    i = pl.program_id(0)

    @pl.when(i == 0)
    def _():
        sums_s[...] = jnp.zeros_like(sums_s)
        cnts_s[...] = jnp.zeros_like(cnts_s)

    dinv = _dinv_block(degb[...])
    acc = accb[0] + accb[1]
    h = jnp.maximum(acc * dinv + bb[...], 0.0)
    oh = (batchb[...] == lax.broadcasted_iota(jnp.int32, (_BM, G), 1))
    oh = oh.astype(jnp.float32)
    sums_s[...] += lax.dot_general(
        oh, h, (((0,), (0,)), ((), ())), preferred_element_type=jnp.float32)
    cnts_s[...] += lax.dot_general(
        oh, jnp.ones((_BM, 1), jnp.float32), (((0,), (0,)), ((), ())),
        preferred_element_type=jnp.float32)

    @pl.when(i == pl.num_programs(0) - 1)
    def _():
        gm = sums_s[...] / jnp.maximum(cnts_s[...], 1.0)
        gm_out[...] = gm
        y_out[...] = jnp.dot(gm, wlb[...],
                             preferred_element_type=jnp.float32) + blb[...]


def _tc_final(accp, degp, b_row, wl, bl_row, batch2d):
    return pl.pallas_call(
        _tc_final_body,
        grid=(_GRID,),
        in_specs=[
            pl.BlockSpec((NC, _BM, H), lambda i: (0, i, 0)),
            pl.BlockSpec((NC, _BM, H), lambda i: (0, i, 0)),
            pl.BlockSpec((1, H), lambda i: (0, 0)),
            pl.BlockSpec((H, C), lambda i: (0, 0)),
            pl.BlockSpec((1, C), lambda i: (0, 0)),
            pl.BlockSpec((_BM, 1), lambda i: (i, 0)),
        ],
        out_specs=[
            pl.BlockSpec((G, C), lambda i: (0, 0)),
            pl.BlockSpec((G, H), lambda i: (0, 0)),
        ],
        out_shape=[
            jax.ShapeDtypeStruct((G, C), jnp.float32),
            jax.ShapeDtypeStruct((G, H), jnp.float32),
        ],
        scratch_shapes=[
            pltpu.VMEM((G, H), jnp.float32),
            pltpu.VMEM((G, 1), jnp.float32),
        ],
    )(accp, degp, b_row, wl, bl_row, batch2d)


def kernel(x, edge_index, batch, W0, b0, W1, b1, W2, b2, Wl, bl):
    x_pad = jnp.pad(x, ((0, NPAD - N), (0, 0)))
    loops = jnp.arange(N, dtype=jnp.int32)
    padv = N + (jnp.arange(E_PAD - E_ALL, dtype=jnp.int32) % (NPAD - N))
    tail = jnp.concatenate([loops, padv])
    pk_edges = edge_index[0] | (edge_index[1] << 16)
    pk_blk = jnp.concatenate(
        [pk_edges, tail | (tail << 16)]).reshape(NW, GPT, EG)
    batch2d = jnp.pad(batch, (0, NPAD - N), constant_values=G).reshape(NPAD, 1)
    b0r = b0.reshape(1, H)
    b1r = b1.reshape(1, H)
    b2r = b2.reshape(1, H)
    blr = bl.reshape(1, C)

    deg_kernel, prop_kernel = _sc_kernels()
    t0 = _tc_matmul(x_pad, W0)
    degp = deg_kernel(pk_blk)

    table = _tc_scale(t0, degp)
    accp = prop_kernel(table, pk_blk)
    table = _tc_mid(accp, degp, b0r, W1)
    accp = prop_kernel(table, pk_blk)
    table = _tc_mid(accp, degp, b1r, W2)
    accp = prop_kernel(table, pk_blk)
    y, gm = _tc_final(accp, degp, b2r, Wl, blr, batch2d)
    return (y, gm)


# speedup vs baseline: 8.2087x; 1.0190x over previous
import functools

import jax
import jax.numpy as jnp
from jax import lax
from jax.experimental import pallas as pl
from jax.experimental.pallas import tpu as pltpu
from jax.experimental.pallas import tpu_sc as plsc

N = 10000
NPAD = 10240
E = 320000
D = 128
H = 128
C = 64
G = 128

NC = 2
NS = 16
NW = NC * NS
EG = 128
E_ALL = E + N
GPT = 82
E_PAD = NW * EG * GPT
ROWS_PER_SUB = NPAD // NS


def _zero_vmem_rows(buf, nrows, width):
    z = jnp.zeros((16,), jnp.float32)

    def body(i, _):
        for j in range(width // 16):
            buf[i, pl.ds(j * 16, 16)] = z
        return 0

    lax.fori_loop(0, nrows, body, 0)


def _fill_vmem_rows(buf, nrows, width, value):
    v = jnp.full((16,), value, jnp.float32)

    def body(i, _):
        for j in range(width // 16):
            buf[i, pl.ds(j * 16, 16)] = v
        return 0

    lax.fori_loop(0, nrows, body, 0)


def _copy_rows_to_shared(buf, acc_sh, base):
    full, rem = divmod(ROWS_PER_SUB, EG)
    for g in range(full):
        pltpu.sync_copy(buf, acc_sh.at[pl.ds(base + g * EG, EG)])
    if rem:
        pltpu.sync_copy(buf.at[pl.ds(0, rem)],
                        acc_sh.at[pl.ds(base + full * EG, rem)])


def _unpack_group(pk_v, j, sbuf, dbuf):
    for k in range(EG // 16):
        v = pk_v[j, pl.ds(k * 16, 16)]
        if sbuf is not None:
            sbuf[pl.ds(k * 16, 16)] = v & 0xFFFF
        dbuf[pl.ds(k * 16, 16)] = v >> 16


def _deg_body(pk_hbm, out_hbm, pk_v, ones_v, dbuf, acc_sh):
    c = lax.axis_index("c")
    s = lax.axis_index("s")
    wid = s * NC + c

    _zero_vmem_rows(ones_v, EG, H)
    _copy_rows_to_shared(ones_v, acc_sh, s * ROWS_PER_SUB)
    _fill_vmem_rows(ones_v, EG, H, 1.0)
    plsc.subcore_barrier()

    pltpu.sync_copy(pk_hbm.at[wid], pk_v)

    def body(j, _):
        _unpack_group(pk_v, j, None, dbuf)
        pltpu.sync_copy(ones_v, acc_sh.at[dbuf], add=True)
        return 0

    lax.fori_loop(0, GPT, body, 0)
    plsc.subcore_barrier()

    pltpu.sync_copy(
        acc_sh.at[pl.ds(s * ROWS_PER_SUB, ROWS_PER_SUB)],
        out_hbm.at[c, pl.ds(s * ROWS_PER_SUB, ROWS_PER_SUB)],
    )


def _prop_body(table_hbm, pk_hbm, out_hbm, pk_v, sa, da, sb, db, rows0,
               rows1, acc_sh, semA, semB):
    c = lax.axis_index("c")
    s = lax.axis_index("s")
    wid = s * NC + c

    _zero_vmem_rows(rows0, EG, H)
    _copy_rows_to_shared(rows0, acc_sh, s * ROWS_PER_SUB)
    plsc.subcore_barrier()

    pltpu.sync_copy(pk_hbm.at[wid], pk_v)

    _unpack_group(pk_v, 0, sa, da)
    pltpu.async_copy(table_hbm.at[sa], rows0, semA)

    def body(jj, _):
        j0 = 2 * jj
        _unpack_group(pk_v, j0 + 1, sb, db)
        pltpu.async_copy(table_hbm.at[sb], rows1, semB)
        pltpu.make_async_copy(table_hbm.at[pl.ds(0, EG)], rows0, semA).wait()
        pltpu.sync_copy(rows0, acc_sh.at[da], add=True)

        _unpack_group(pk_v, jnp.minimum(j0 + 2, GPT - 1), sa, da)
        pltpu.async_copy(table_hbm.at[sa], rows0, semA)

        pltpu.make_async_copy(table_hbm.at[pl.ds(0, EG)], rows1, semB).wait()
        pltpu.sync_copy(rows1, acc_sh.at[db], add=True)
        return 0

    lax.fori_loop(0, GPT // 2, body, 0)
    pltpu.make_async_copy(table_hbm.at[pl.ds(0, EG)], rows0, semA).wait()
    plsc.subcore_barrier()

    pltpu.sync_copy(
        acc_sh.at[pl.ds(s * ROWS_PER_SUB, ROWS_PER_SUB)],
        out_hbm.at[c, pl.ds(s * ROWS_PER_SUB, ROWS_PER_SUB)],
    )


@functools.cache
def _sc_kernels():
    mesh = plsc.VectorSubcoreMesh(core_axis_name="c", subcore_axis_name="s")
    deg = pl.kernel(
        _deg_body,
        out_type=jax.ShapeDtypeStruct((NC, NPAD, H), jnp.float32),
        mesh=mesh,
        scratch_types=[
            pltpu.VMEM((GPT, EG), jnp.int32),
            pltpu.VMEM((EG, H), jnp.float32),
            pltpu.VMEM((EG,), jnp.int32),
            pltpu.VMEM_SHARED((NPAD, H), jnp.float32),
        ],
    )
    prop = pl.kernel(
        _prop_body,
        out_type=jax.ShapeDtypeStruct((NC, NPAD, H), jnp.float32),
        mesh=mesh,
        scratch_types=[
            pltpu.VMEM((GPT, EG), jnp.int32),
            pltpu.VMEM((EG,), jnp.int32),
            pltpu.VMEM((EG,), jnp.int32),
            pltpu.VMEM((EG,), jnp.int32),
            pltpu.VMEM((EG,), jnp.int32),
            pltpu.VMEM((EG, H), jnp.float32),
            pltpu.VMEM((EG, H), jnp.float32),
            pltpu.VMEM_SHARED((NPAD, H), jnp.float32),
            pltpu.SemaphoreType.DMA,
            pltpu.SemaphoreType.DMA,
        ],
    )
    return deg, prop


_BM = 1024
_GRID = NPAD // _BM
_EROWS = E // EG
_TROWS = (E_PAD - E) // EG


def _tc_pack_body(eb, pkb):
    e = eb[...]
    pk = e[0] | (e[1] << 16)
    p = (E + lax.broadcasted_iota(jnp.int32, (_TROWS, EG), 0) * EG
         + lax.broadcasted_iota(jnp.int32, (_TROWS, EG), 1))
    v = jnp.where(p < E_ALL, p - E, N + (p - E_ALL) % (NPAD - N))
    pkb[...] = jnp.concatenate([pk, v | (v << 16)], axis=0)


def _tc_pack(edge_index):
    return pl.pallas_call(
        _tc_pack_body,
        in_specs=[pl.BlockSpec((2, _EROWS, EG), lambda: (0, 0, 0))],
        out_specs=pl.BlockSpec((_EROWS + _TROWS, EG), lambda: (0, 0)),
        out_shape=jax.ShapeDtypeStruct((_EROWS + _TROWS, EG), jnp.int32),
    )(edge_index.reshape(2, _EROWS, EG))


def _dinv_block(degb):
    deg = degb[0, :, 0:1] + degb[1, :, 0:1]
    return lax.rsqrt(jnp.maximum(deg, 1.0))


def _tc_matmul_body(xb, wb, tb):
    tb[...] = jnp.dot(xb[...], wb[...], preferred_element_type=jnp.float32)


def _tc_matmul(x_pad, w):
    return pl.pallas_call(
        _tc_matmul_body,
        grid=(_GRID,),
        in_specs=[
            pl.BlockSpec((_BM, D), lambda i: (i, 0)),
            pl.BlockSpec((D, H), lambda i: (0, 0)),
        ],
        out_specs=pl.BlockSpec((_BM, H), lambda i: (i, 0)),
        out_shape=jax.ShapeDtypeStruct((NPAD, H), jnp.float32),
    )(x_pad, w)


def _tc_scale_body(tb, degb, tableb, dinvb):
    dinv = _dinv_block(degb[...])
    tableb[...] = tb[...] * dinv
    dinvb[...] = jnp.broadcast_to(dinv, (_BM, H))


def _tc_scale(t0, degp):
    return pl.pallas_call(
        _tc_scale_body,
        grid=(_GRID,),
        in_specs=[
            pl.BlockSpec((_BM, H), lambda i: (i, 0)),
            pl.BlockSpec((NC, _BM, H), lambda i: (0, i, 0)),
        ],
        out_specs=[
            pl.BlockSpec((_BM, H), lambda i: (i, 0)),
            pl.BlockSpec((_BM, H), lambda i: (i, 0)),
        ],
        out_shape=[
            jax.ShapeDtypeStruct((NPAD, H), jnp.float32),
            jax.ShapeDtypeStruct((NPAD, H), jnp.float32),
        ],
    )(t0, degp)


def _tc_mid_body(accb, dinvb, bb, wb, tableb):
    dinv = dinvb[:, 0:1]
    acc = accb[0] + accb[1]
    h = jnp.maximum(acc * dinv + bb[...], 0.0)
    t = jnp.dot(h, wb[...], preferred_element_type=jnp.float32)
    tableb[...] = t * dinv


def _tc_mid(accp, dinv, b_row, w):
    return pl.pallas_call(
        _tc_mid_body,
        grid=(_GRID,),
        in_specs=[
            pl.BlockSpec((NC, _BM, H), lambda i: (0, i, 0)),
            pl.BlockSpec((_BM, H), lambda i: (i, 0)),
            pl.BlockSpec((1, H), lambda i: (0, 0)),
            pl.BlockSpec((H, H), lambda i: (0, 0)),
        ],
        out_specs=pl.BlockSpec((_BM, H), lambda i: (i, 0)),
        out_shape=jax.ShapeDtypeStruct((NPAD, H), jnp.float32),
    )(accp, dinv, b_row, w)


def _tc_final_body(accb, dinvb, bb, wlb, blb, batchb, y_out, gm_out,
                   sums_s, cnts_s):
    i = pl.program_id(0)

    @pl.when(i == 0)
    def _():
        sums_s[...] = jnp.zeros_like(sums_s)
        cnts_s[...] = jnp.zeros_like(cnts_s)

    dinv = dinvb[:, 0:1]
    acc = accb[0] + accb[1]
    h = jnp.maximum(acc * dinv + bb[...], 0.0)
    oh = (batchb[...] == lax.broadcasted_iota(jnp.int32, (_BM, G), 1))
    oh = oh.astype(jnp.float32)
    sums_s[...] += lax.dot_general(
        oh, h, (((0,), (0,)), ((), ())), preferred_element_type=jnp.float32)
    cnts_s[...] += lax.dot_general(
        oh, jnp.ones((_BM, 1), jnp.float32), (((0,), (0,)), ((), ())),
        preferred_element_type=jnp.float32)

    @pl.when(i == pl.num_programs(0) - 1)
    def _():
        gm = sums_s[...] / jnp.maximum(cnts_s[...], 1.0)
        gm_out[...] = gm
        y_out[...] = jnp.dot(gm, wlb[...],
                             preferred_element_type=jnp.float32) + blb[...]


def _tc_final(accp, dinv, b_row, wl, bl_row, batch2d):
    return pl.pallas_call(
        _tc_final_body,
        grid=(_GRID,),
        in_specs=[
            pl.BlockSpec((NC, _BM, H), lambda i: (0, i, 0)),
            pl.BlockSpec((_BM, H), lambda i: (i, 0)),
            pl.BlockSpec((1, H), lambda i: (0, 0)),
            pl.BlockSpec((H, C), lambda i: (0, 0)),
            pl.BlockSpec((1, C), lambda i: (0, 0)),
            pl.BlockSpec((_BM, 1), lambda i: (i, 0)),
        ],
        out_specs=[
            pl.BlockSpec((G, C), lambda i: (0, 0)),
            pl.BlockSpec((G, H), lambda i: (0, 0)),
        ],
        out_shape=[
            jax.ShapeDtypeStruct((G, C), jnp.float32),
            jax.ShapeDtypeStruct((G, H), jnp.float32),
        ],
        scratch_shapes=[
            pltpu.VMEM((G, H), jnp.float32),
            pltpu.VMEM((G, 1), jnp.float32),
        ],
    )(accp, dinv, b_row, wl, bl_row, batch2d)


def kernel(x, edge_index, batch, W0, b0, W1, b1, W2, b2, Wl, bl):
    x_pad = jnp.pad(x, ((0, NPAD - N), (0, 0)))
    pk_blk = _tc_pack(edge_index).reshape(NW, GPT, EG)
    batch2d = jnp.pad(batch, (0, NPAD - N), constant_values=G).reshape(NPAD, 1)
    b0r = b0.reshape(1, H)
    b1r = b1.reshape(1, H)
    b2r = b2.reshape(1, H)
    blr = bl.reshape(1, C)

    deg_kernel, prop_kernel = _sc_kernels()
    t0 = _tc_matmul(x_pad, W0)
    degp = deg_kernel(pk_blk)

    table, dinv = _tc_scale(t0, degp)
    accp = prop_kernel(table, pk_blk)
    table = _tc_mid(accp, dinv, b0r, W1)
    accp = prop_kernel(table, pk_blk)
    table = _tc_mid(accp, dinv, b1r, W2)
    accp = prop_kernel(table, pk_blk)
    y, gm = _tc_final(accp, dinv, b2r, Wl, blr, batch2d)
    return (y, gm)
